# Initial kernel scaffold; baseline (speedup 1.0000x reference)
import jax
import jax.numpy as jnp
from jax.experimental import pallas as pl


def kernel(x, edge_index, edge_weight, kernel, self_kernel, self_loop_weight, bias):
    raise NotImplementedError("write your pallas kernel here")


# speedup vs baseline: 3.7968x; 3.7968x over previous
import functools

import jax
import jax.numpy as jnp
from jax import lax
from jax.experimental import pallas as pl
from jax.experimental.pallas import tpu as pltpu
from jax.experimental.pallas import tpu_sc as plsc

N = 10000
E = 320000
D = 128
L = 16
NC = 2
NS = 16
NW = NC * NS
CHUNK = 128
EPAD = 323584
CPW = EPAD // (NW * CHUNK)
NPAD = 10240
ROWS_PER_TILE = NPAD // NS
MM_BLOCK = 2000


def _mm_body(x_ref, k_ref, sk_ref, b_ref, h_ref, z_ref):
    xb = x_ref[...]
    h_ref[...] = jnp.dot(xb, k_ref[...], preferred_element_type=jnp.float32)
    z_ref[...] = (
        jnp.dot(xb, sk_ref[...], preferred_element_type=jnp.float32)
        + b_ref[...]
    )


def _matmuls(x2d, w, sw, bias):
    grid = N // MM_BLOCK
    return pl.pallas_call(
        _mm_body,
        grid=(grid,),
        in_specs=[
            pl.BlockSpec((MM_BLOCK, D), lambda i: (i, 0)),
            pl.BlockSpec((D, D), lambda i: (0, 0)),
            pl.BlockSpec((D, D), lambda i: (0, 0)),
            pl.BlockSpec((D,), lambda i: (0,)),
        ],
        out_specs=[
            pl.BlockSpec((MM_BLOCK, D), lambda i: (i, 0)),
            pl.BlockSpec((MM_BLOCK, D), lambda i: (i, 0)),
        ],
        out_shape=[
            jax.ShapeDtypeStruct((N, D), jnp.float32),
            jax.ShapeDtypeStruct((N, D), jnp.float32),
        ],
    )(x2d, w, sw, bias)


def _fin_body(z_ref, p0_ref, p1_ref, o_ref):
    o_ref[...] = jnp.maximum(z_ref[...] + p0_ref[...] + p1_ref[...], 0.0)


def _finalize(z, p0, p1):
    grid = N // MM_BLOCK
    spec = pl.BlockSpec((MM_BLOCK, D), lambda i: (i, 0))
    return pl.pallas_call(
        _fin_body,
        grid=(grid,),
        in_specs=[spec, spec, spec],
        out_specs=spec,
        out_shape=jax.ShapeDtypeStruct((N, D), jnp.float32),
    )(z, p0, p1)


def _sc_aggregate_body(h_hbm, src_hbm, dst_hbm, ew_hbm, p0_hbm, p1_hbm,
                       src_v, dst_v, ew_v, rows_v, acc, sem):
    c = lax.axis_index("c")
    s = lax.axis_index("s")
    wid = c * NS + s

    pltpu.sync_copy(src_hbm.at[wid], src_v)
    pltpu.sync_copy(dst_hbm.at[wid], dst_v)
    pltpu.sync_copy(ew_hbm.at[wid], ew_v)

    def _zrow(r, carry):
        for f in range(D // L):
            rows_v[r, pl.ds(f * L, L)] = jnp.zeros((L,), jnp.float32)
        return carry

    lax.fori_loop(0, CHUNK, _zrow, 0)
    for k in range(ROWS_PER_TILE // CHUNK):
        pltpu.sync_copy(
            rows_v,
            acc.at[pl.ds(s * ROWS_PER_TILE + k * CHUNK, CHUNK)],
        )
    plsc.subcore_barrier()

    def _chunk(j, carry):
        pltpu.async_copy(h_hbm.at[src_v.at[j]], rows_v, sem).wait()

        def _row(r, rcarry):
            w = plsc.load_gather(
                ew_v, [jnp.full((L,), j * CHUNK + r, jnp.int32)]
            )
            for f in range(D // L):
                sl = pl.ds(f * L, L)
                rows_v[r, sl] = rows_v[r, sl] * w
            return rcarry

        lax.fori_loop(0, CHUNK, _row, 0)
        pltpu.sync_copy(rows_v, acc.at[dst_v.at[j]], add=True)
        return carry

    lax.fori_loop(0, CPW, _chunk, 0)
    plsc.subcore_barrier()

    @pl.when(c == 0)
    def _():
        pltpu.sync_copy(
            acc.at[pl.ds(s * ROWS_PER_TILE, ROWS_PER_TILE)],
            p0_hbm.at[pl.ds(s * ROWS_PER_TILE, ROWS_PER_TILE)],
        )

    @pl.when(c == 1)
    def _():
        pltpu.sync_copy(
            acc.at[pl.ds(s * ROWS_PER_TILE, ROWS_PER_TILE)],
            p1_hbm.at[pl.ds(s * ROWS_PER_TILE, ROWS_PER_TILE)],
        )


@functools.cache
def _sc_aggregate():
    return pl.kernel(
        _sc_aggregate_body,
        out_type=(
            jax.ShapeDtypeStruct((NPAD, D), jnp.float32),
            jax.ShapeDtypeStruct((NPAD, D), jnp.float32),
        ),
        mesh=plsc.VectorSubcoreMesh(
            core_axis_name="c", subcore_axis_name="s",
            num_cores=NC, num_subcores=NS,
        ),
        scratch_types=[
            pltpu.VMEM((CPW, CHUNK), jnp.int32),
            pltpu.VMEM((CPW, CHUNK), jnp.int32),
            pltpu.VMEM((CPW * CHUNK,), jnp.float32),
            pltpu.VMEM((CHUNK, D), jnp.float32),
            pltpu.VMEM_SHARED((NPAD, D), jnp.float32),
            pltpu.SemaphoreType.DMA,
        ],
        compiler_params=pltpu.CompilerParams(needs_layout_passes=False),
    )


def kernel(x, edge_index, edge_weight, kernel, self_kernel,
           self_loop_weight, bias):
    x2d = jnp.squeeze(x, axis=0)
    sk_scaled = self_kernel * self_loop_weight[0]
    h, z = _matmuls(x2d, kernel, sk_scaled, bias)

    pad = EPAD - E
    src = jnp.concatenate(
        [edge_index[0].astype(jnp.int32), jnp.zeros((pad,), jnp.int32)]
    ).reshape(NW, CPW, CHUNK)
    dst = jnp.concatenate(
        [edge_index[1].astype(jnp.int32), jnp.zeros((pad,), jnp.int32)]
    ).reshape(NW, CPW, CHUNK)
    ew = jnp.concatenate(
        [edge_weight.astype(jnp.float32), jnp.zeros((pad,), jnp.float32)]
    ).reshape(NW, CPW * CHUNK)

    p0, p1 = _sc_aggregate()(h, src, dst, ew)
    out = _finalize(z, p0, p1)
    return out[None, :, :]
